# Initial kernel scaffold; baseline (speedup 1.0000x reference)
import functools

import jax
import jax.numpy as jnp
from jax import lax
from jax.experimental import pallas as pl
from jax.experimental.pallas import tpu as pltpu
from jax.experimental.pallas import tpu_sc as plsc

N = 10000
E = 320000
R = 8
D = 128
C = 40
CP = 48
NR = N * R

NC = 2
NS = 16
NW = NC * NS
EPW = E // NW
ECH = 80
NCHUNK = EPW // ECH
RPT = NR // NS
NPT = N // NS
ZCH = 125

_mesh = plsc.VectorSubcoreMesh(core_axis_name="c", subcore_axis_name="s")
_f32 = jnp.float32
_i32 = jnp.int32


def _hist_body(dst_h, et_h, zeros_h, ones_h, part_h,
               dstv, etv, idxv, onesv, zv, cnt_sh):
    cid = lax.axis_index("c")
    sid = lax.axis_index("s")
    a0 = sid * RPT
    pltpu.sync_copy(zeros_h, zv)
    pltpu.sync_copy(zv, cnt_sh.at[pl.ds(a0, RPT)])
    pltpu.sync_copy(ones_h, onesv)
    plsc.subcore_barrier()

    base = (cid * NS + sid) * EPW

    def blk(b, carry):
        off = base + b * ECH
        pltpu.sync_copy(dst_h.at[pl.ds(off, ECH)], dstv)
        pltpu.sync_copy(et_h.at[pl.ds(off, ECH)], etv)
        for j in range(ECH // 16):
            d16 = dstv[pl.ds(j * 16, 16)]
            t16 = etv[pl.ds(j * 16, 16)]
            idxv[pl.ds(j * 16, 16)] = d16 * R + t16
        pltpu.sync_copy(onesv, cnt_sh.at[idxv], add=True)
        return carry

    lax.fori_loop(0, NCHUNK, blk, 0)
    plsc.subcore_barrier()
    pltpu.sync_copy(cnt_sh.at[pl.ds(a0, RPT)], zv)
    pltpu.sync_copy(zv, part_h.at[cid, pl.ds(a0, RPT)])


_hist = pl.kernel(
    _hist_body,
    out_type=jax.ShapeDtypeStruct((NC, NR, 1), _f32),
    mesh=_mesh,
    scratch_types=[
        pltpu.VMEM((ECH,), _i32),
        pltpu.VMEM((ECH,), _i32),
        pltpu.VMEM((ECH,), _i32),
        pltpu.VMEM((ECH, 1), _f32),
        pltpu.VMEM((RPT, 1), _f32),
        pltpu.VMEM_SHARED((NR, 1), _f32),
    ],
)


def _prep_body(src_h, dst_h, et_h, part_h, w_h, rid_h,
               av, bv, cv, invv, srcv, dstv, etv, wbuf, ridbuf, inv_sh):
    cid = lax.axis_index("c")
    sid = lax.axis_index("s")
    a0 = sid * RPT
    pltpu.sync_copy(part_h.at[0, pl.ds(a0, RPT)], av)
    pltpu.sync_copy(part_h.at[1, pl.ds(a0, RPT)], bv)

    def inv_blk(k, carry):
        off = jnp.minimum(k * 16, RPT - 16)
        cnt16 = av[pl.ds(off, 16)] + bv[pl.ds(off, 16)]
        cv[pl.ds(off, 16)] = 1.0 / jnp.maximum(cnt16, 1.0)
        return carry

    lax.fori_loop(0, (RPT + 15) // 16, inv_blk, 0)
    pltpu.sync_copy(cv, inv_sh.at[pl.ds(a0, RPT)])
    plsc.subcore_barrier()
    pltpu.sync_copy(inv_sh, invv)

    base = (cid * NS + sid) * EPW

    def blk(b, carry):
        off = base + b * ECH
        pltpu.sync_copy(src_h.at[pl.ds(off, ECH)], srcv)
        pltpu.sync_copy(dst_h.at[pl.ds(off, ECH)], dstv)
        pltpu.sync_copy(et_h.at[pl.ds(off, ECH)], etv)
        for j in range(ECH // 16):
            s16 = srcv[pl.ds(j * 16, 16)]
            d16 = dstv[pl.ds(j * 16, 16)]
            t16 = etv[pl.ds(j * 16, 16)]
            seg = d16 * R + t16
            wbuf[pl.ds(j * 16, 16)] = plsc.load_gather(invv, [seg])
            ridbuf[pl.ds(j * 16, 16)] = t16 * N + s16
        pltpu.sync_copy(wbuf, w_h.at[pl.ds(off, ECH)])
        pltpu.sync_copy(ridbuf, rid_h.at[pl.ds(off, ECH)])
        return carry

    lax.fori_loop(0, NCHUNK, blk, 0)


_prep = pl.kernel(
    _prep_body,
    out_type=(jax.ShapeDtypeStruct((E,), _f32),
              jax.ShapeDtypeStruct((E,), _i32)),
    mesh=_mesh,
    scratch_types=[
        pltpu.VMEM((RPT,), _f32),
        pltpu.VMEM((RPT,), _f32),
        pltpu.VMEM((RPT,), _f32),
        pltpu.VMEM((NR,), _f32),
        pltpu.VMEM((ECH,), _i32),
        pltpu.VMEM((ECH,), _i32),
        pltpu.VMEM((ECH,), _i32),
        pltpu.VMEM((ECH,), _f32),
        pltpu.VMEM((ECH,), _i32),
        pltpu.VMEM_SHARED((NR,), _f32),
    ],
)


def _edge_body(F, rid_h, dst_h, w_h, y_h, zeros_h, acc_h,
               ridv, didxv, wv, rowsv, zv, sem, acc_sh):
    cid = lax.axis_index("c")
    sid = lax.axis_index("s")
    pltpu.sync_copy(zeros_h, zv)
    for k in range(NPT // ZCH):
        pltpu.sync_copy(zv, acc_sh.at[pl.ds(sid * NPT + k * ZCH, ZCH)])
    plsc.subcore_barrier()

    base = (cid * NS + sid) * EPW

    def blk(b, carry):
        off = base + b * ECH
        pltpu.sync_copy(rid_h.at[pl.ds(off, ECH)], ridv)
        pltpu.sync_copy(dst_h.at[pl.ds(off, ECH)], didxv)
        pltpu.sync_copy(w_h.at[pl.ds(off, ECH)], wv)
        pltpu.async_copy(y_h.at[ridv], rowsv, sem).wait()

        def ei(i, c2):
            ws = plsc.load_gather(wv, [jnp.full((16,), i, _i32)])
            for j in range(F // 16):
                rowsv[i, pl.ds(j * 16, 16)] = rowsv[i, pl.ds(j * 16, 16)] * ws
            return c2

        lax.fori_loop(0, ECH, ei, 0)
        pltpu.sync_copy(rowsv, acc_sh.at[didxv], add=True)
        return carry

    lax.fori_loop(0, NCHUNK, blk, 0)
    plsc.subcore_barrier()
    for k in range(NPT // ZCH):
        a = sid * NPT + k * ZCH
        pltpu.sync_copy(acc_sh.at[pl.ds(a, ZCH)], zv)
        pltpu.sync_copy(zv, acc_h.at[cid, pl.ds(a, ZCH)])


def _make_edge(F):
    return pl.kernel(
        functools.partial(_edge_body, F),
        out_type=jax.ShapeDtypeStruct((NC, N, F), _f32),
        mesh=_mesh,
        scratch_types=[
            pltpu.VMEM((ECH,), _i32),
            pltpu.VMEM((ECH,), _i32),
            pltpu.VMEM((ECH,), _f32),
            pltpu.VMEM((ECH, F), _f32),
            pltpu.VMEM((ZCH, F), _f32),
            pltpu.SemaphoreType.DMA,
            pltpu.VMEM_SHARED((N, F), _f32),
        ],
    )


_edge_l1 = _make_edge(D)
_edge_l2 = _make_edge(CP)


BN = 1000
NB = N // BN


def _mm_body(x_ref, w_ref, o_ref):
    o_ref[...] = jnp.dot(x_ref[...], w_ref[0],
                         preferred_element_type=jnp.float32)


def _mm(x, ws):
    nmat, _, f = ws.shape
    return pl.pallas_call(
        _mm_body,
        grid=(nmat, NB),
        in_specs=[
            pl.BlockSpec((BN, D), lambda r, i: (i, 0)),
            pl.BlockSpec((1, D, f), lambda r, i: (r, 0, 0)),
        ],
        out_specs=pl.BlockSpec((BN, f), lambda r, i: (r * NB + i, 0)),
        out_shape=jax.ShapeDtypeStruct((nmat * N, f), _f32),
    )(x, ws)


def _l2_body(acc_ref, y0_ref, b_ref, w_ref, o_ref):
    h = jnp.maximum(acc_ref[0] + acc_ref[1] + y0_ref[...] + b_ref[...], 0.0)
    o_ref[...] = jnp.dot(h, w_ref[0], preferred_element_type=jnp.float32)


def _l2(acc, y0, b, ws):
    nmat = ws.shape[0]
    return pl.pallas_call(
        _l2_body,
        grid=(nmat, NB),
        in_specs=[
            pl.BlockSpec((2, BN, D), lambda r, i: (0, i, 0)),
            pl.BlockSpec((BN, D), lambda r, i: (R * NB + i, 0)),
            pl.BlockSpec((1, D), lambda r, i: (0, 0)),
            pl.BlockSpec((1, D, CP), lambda r, i: (r, 0, 0)),
        ],
        out_specs=pl.BlockSpec((BN, CP), lambda r, i: (r * NB + i, 0)),
        out_shape=jax.ShapeDtypeStruct((nmat * N, CP), _f32),
    )(acc, y0, b, ws)


def _out_body(acc_ref, y1_ref, b_ref, o_ref):
    z = jnp.maximum(acc_ref[0] + acc_ref[1] + y1_ref[...] + b_ref[...], 0.0)
    col = lax.broadcasted_iota(_i32, z.shape, 1)
    msk = col < C
    zm = jnp.where(msk, z, -jnp.inf)
    m = jnp.max(zm, axis=1, keepdims=True)
    s = jnp.sum(jnp.where(msk, jnp.exp(z - m), 0.0), axis=1, keepdims=True)
    o_ref[...] = z - m - jnp.log(s)


def _outk(acc, y1, b):
    return pl.pallas_call(
        _out_body,
        grid=(NB,),
        in_specs=[
            pl.BlockSpec((2, BN, CP), lambda i: (0, i, 0)),
            pl.BlockSpec((BN, CP), lambda i: (R * NB + i, 0)),
            pl.BlockSpec((1, CP), lambda i: (0, 0)),
        ],
        out_specs=pl.BlockSpec((BN, CP), lambda i: (i, 0)),
        out_shape=jax.ShapeDtypeStruct((N, CP), _f32),
    )(acc, y1, b)


def kernel(x, edge_index, edge_type, W_rel0, W_root0, b0, W_rel1, W_root1, b1):
    src = edge_index[0]
    dst = edge_index[1]
    et = edge_type

    zeros_nr = jnp.zeros((RPT, 1), _f32)
    ones_e = jnp.ones((ECH, 1), _f32)
    part = _hist(dst, et, zeros_nr, ones_e)
    w_e, rid = _prep(src, dst, et, part.reshape(NC, NR))

    W0s = jnp.concatenate([W_rel0, W_root0[None]], axis=0)
    y0 = _mm(x, W0s)
    zeros_d = jnp.zeros((ZCH, D), _f32)
    acc0 = _edge_l1(rid, dst, w_e, y0, zeros_d)

    W1s = jnp.concatenate(
        [jnp.pad(W_rel1, ((0, 0), (0, 0), (0, CP - C))),
         jnp.pad(W_root1, ((0, 0), (0, CP - C)))[None]], axis=0)
    y1 = _l2(acc0, y0, b0.reshape(1, D), W1s)
    zeros_c = jnp.zeros((ZCH, CP), _f32)
    acc1 = _edge_l2(rid, dst, w_e, y1, zeros_c)

    b1p = jnp.pad(b1, (0, CP - C)).reshape(1, CP)
    outp = _outk(acc1, y1, b1p)
    return outp[:, :C]


# speedup vs baseline: 2.3215x; 2.3215x over previous
import functools

import jax
import jax.numpy as jnp
from jax import lax
from jax.experimental import pallas as pl
from jax.experimental.pallas import tpu as pltpu
from jax.experimental.pallas import tpu_sc as plsc

N = 10000
E = 320000
R = 8
D = 128
C = 40
CP = 48
NR = N * R

NC = 2
NS = 16
NW = NC * NS
EPW = E // NW
ECH = 80
NCHUNK = EPW // ECH
RPT = NR // NS
NPT = N // NS
ZCH = 125

_mesh = plsc.VectorSubcoreMesh(core_axis_name="c", subcore_axis_name="s",
                               num_cores=NC, num_subcores=NS)
_f32 = jnp.float32
_i32 = jnp.int32


ROWW = 16


def _hist_body(dst_h, et_h, zeros_h, ones_h, part_h,
               dstv, etv, idxv, onesv, zv, cnt_sh):
    cid = lax.axis_index("c")
    sid = lax.axis_index("s")
    a0 = sid * RPT
    pltpu.sync_copy(zeros_h, zv)
    for k in range(RPT // ZRT):
        pltpu.sync_copy(zv, cnt_sh.at[pl.ds(a0 + k * ZRT, ZRT)])
    pltpu.sync_copy(ones_h, onesv)
    plsc.subcore_barrier()

    base = (cid * NS + sid) * EPW

    def blk(b, carry):
        off = base + b * ECH
        pltpu.sync_copy(dst_h.at[pl.ds(off, ECH)], dstv)
        pltpu.sync_copy(et_h.at[pl.ds(off, ECH)], etv)
        for j in range(ECH // 16):
            d16 = dstv[pl.ds(j * 16, 16)]
            t16 = etv[pl.ds(j * 16, 16)]
            idxv[pl.ds(j * 16, 16)] = d16 * R + t16
        pltpu.sync_copy(onesv, cnt_sh.at[idxv], add=True)
        return carry

    lax.fori_loop(0, NCHUNK, blk, 0)
    plsc.subcore_barrier()
    for k in range(RPT // ZRT):
        pltpu.sync_copy(cnt_sh.at[pl.ds(a0 + k * ZRT, ZRT), pl.ds(0, 1)],
                        zv.at[pl.ds(0, ZRT), pl.ds(0, 1)])
        pltpu.sync_copy(zv.at[pl.ds(0, ZRT), pl.ds(0, 1)],
                        part_h.at[cid, pl.ds(a0 + k * ZRT, ZRT)])


ZRT = 1000

_hist = pl.kernel(
    _hist_body,
    out_type=jax.ShapeDtypeStruct((NC, NR, 1), _f32),
    mesh=_mesh,
    compiler_params=pltpu.CompilerParams(use_tc_tiling_on_sc=False, needs_layout_passes=False),
    scratch_types=[
        pltpu.VMEM((ECH,), _i32),
        pltpu.VMEM((ECH,), _i32),
        pltpu.VMEM((ECH,), _i32),
        pltpu.VMEM((ECH, ROWW), _f32),
        pltpu.VMEM((ZRT, ROWW), _f32),
        pltpu.VMEM_SHARED((NR, ROWW), _f32),
    ],
)


def _prep_body(src_h, dst_h, et_h, part_h, w_h, rid_h,
               av, bv, cv, invv, srcv, dstv, etv, wbuf, ridbuf, inv_sh):
    cid = lax.axis_index("c")
    sid = lax.axis_index("s")
    a0 = sid * RPT
    pltpu.sync_copy(part_h.at[0, pl.ds(a0, RPT)], av)
    pltpu.sync_copy(part_h.at[1, pl.ds(a0, RPT)], bv)

    def inv_blk(k, carry):
        off = jnp.minimum(k * 16, RPT - 16)
        cnt16 = av[pl.ds(off, 16)] + bv[pl.ds(off, 16)]
        cv[pl.ds(off, 16)] = 1.0 / jnp.maximum(cnt16, 1.0)
        return carry

    lax.fori_loop(0, (RPT + 15) // 16, inv_blk, 0)
    pltpu.sync_copy(cv, inv_sh.at[pl.ds(a0, RPT)])
    plsc.subcore_barrier()
    pltpu.sync_copy(inv_sh, invv)

    base = (cid * NS + sid) * EPW

    def blk(b, carry):
        off = base + b * ECH
        pltpu.sync_copy(src_h.at[pl.ds(off, ECH)], srcv)
        pltpu.sync_copy(dst_h.at[pl.ds(off, ECH)], dstv)
        pltpu.sync_copy(et_h.at[pl.ds(off, ECH)], etv)
        for j in range(ECH // 16):
            s16 = srcv[pl.ds(j * 16, 16)]
            d16 = dstv[pl.ds(j * 16, 16)]
            t16 = etv[pl.ds(j * 16, 16)]
            seg = d16 * R + t16
            wbuf[pl.ds(j * 16, 16)] = plsc.load_gather(invv, [seg])
            ridbuf[pl.ds(j * 16, 16)] = t16 * N + s16
        pltpu.sync_copy(wbuf, w_h.at[pl.ds(off, ECH)])
        pltpu.sync_copy(ridbuf, rid_h.at[pl.ds(off, ECH)])
        return carry

    lax.fori_loop(0, NCHUNK, blk, 0)


_prep = pl.kernel(
    _prep_body,
    out_type=(jax.ShapeDtypeStruct((E,), _f32),
              jax.ShapeDtypeStruct((E,), _i32)),
    mesh=_mesh,
    compiler_params=pltpu.CompilerParams(use_tc_tiling_on_sc=False, needs_layout_passes=False),
    scratch_types=[
        pltpu.VMEM((RPT,), _f32),
        pltpu.VMEM((RPT,), _f32),
        pltpu.VMEM((RPT,), _f32),
        pltpu.VMEM((NR,), _f32),
        pltpu.VMEM((ECH,), _i32),
        pltpu.VMEM((ECH,), _i32),
        pltpu.VMEM((ECH,), _i32),
        pltpu.VMEM((ECH,), _f32),
        pltpu.VMEM((ECH,), _i32),
        pltpu.VMEM_SHARED((NR,), _f32),
    ],
)


def _edge_body(F, rid_h, dst_h, w_h, y_h, zeros_h, acc_h,
               ridv, didxv, wv, rowsv, zv, sem, acc_sh):
    cid = lax.axis_index("c")
    sid = lax.axis_index("s")
    pltpu.sync_copy(zeros_h, zv)
    for k in range(NPT // ZCH):
        pltpu.sync_copy(zv, acc_sh.at[pl.ds(sid * NPT + k * ZCH, ZCH)])
    plsc.subcore_barrier()

    base = (cid * NS + sid) * EPW

    def blk(b, carry):
        off = base + b * ECH
        pltpu.sync_copy(rid_h.at[pl.ds(off, ECH)], ridv)
        pltpu.sync_copy(dst_h.at[pl.ds(off, ECH)], didxv)
        pltpu.sync_copy(w_h.at[pl.ds(off, ECH)], wv)
        pltpu.async_copy(y_h.at[ridv], rowsv, sem).wait()

        def ei(i, c2):
            ws = plsc.load_gather(wv, [jnp.full((16,), i, _i32)])
            for j in range(F // 16):
                rowsv[i, pl.ds(j * 16, 16)] = rowsv[i, pl.ds(j * 16, 16)] * ws
            return c2

        lax.fori_loop(0, ECH, ei, 0)
        pltpu.sync_copy(rowsv, acc_sh.at[didxv], add=True)
        return carry

    lax.fori_loop(0, NCHUNK, blk, 0)
    plsc.subcore_barrier()
    for k in range(NPT // ZCH):
        a = sid * NPT + k * ZCH
        pltpu.sync_copy(acc_sh.at[pl.ds(a, ZCH)], zv)
        pltpu.sync_copy(zv, acc_h.at[cid, pl.ds(a, ZCH)])


def _make_edge(F):
    return pl.kernel(
        functools.partial(_edge_body, F),
        out_type=jax.ShapeDtypeStruct((NC, N, F), _f32),
        mesh=_mesh,
        compiler_params=pltpu.CompilerParams(use_tc_tiling_on_sc=False, needs_layout_passes=False),
        scratch_types=[
            pltpu.VMEM((ECH,), _i32),
            pltpu.VMEM((ECH,), _i32),
            pltpu.VMEM((ECH,), _f32),
            pltpu.VMEM((ECH, F), _f32),
            pltpu.VMEM((ZCH, F), _f32),
            pltpu.SemaphoreType.DMA,
            pltpu.VMEM_SHARED((N, F), _f32),
        ],
    )


_edge_l1 = _make_edge(D)
_edge_l2 = _make_edge(CP)


BN = 1000
NB = N // BN


def _mm_body(x_ref, w_ref, o_ref):
    o_ref[...] = jnp.dot(x_ref[...], w_ref[0],
                         preferred_element_type=jnp.float32)


def _mm(x, ws):
    nmat, _, f = ws.shape
    return pl.pallas_call(
        _mm_body,
        grid=(nmat, NB),
        in_specs=[
            pl.BlockSpec((BN, D), lambda r, i: (i, 0)),
            pl.BlockSpec((1, D, f), lambda r, i: (r, 0, 0)),
        ],
        out_specs=pl.BlockSpec((BN, f), lambda r, i: (r * NB + i, 0)),
        out_shape=jax.ShapeDtypeStruct((nmat * N, f), _f32),
    )(x, ws)


def _l2_body(acc_ref, y0_ref, b_ref, w_ref, o_ref):
    h = jnp.maximum(acc_ref[0] + acc_ref[1] + y0_ref[...] + b_ref[...], 0.0)
    o_ref[...] = jnp.dot(h, w_ref[0], preferred_element_type=jnp.float32)


def _l2(acc, y0, b, ws):
    nmat = ws.shape[0]
    return pl.pallas_call(
        _l2_body,
        grid=(nmat, NB),
        in_specs=[
            pl.BlockSpec((2, BN, D), lambda r, i: (0, i, 0)),
            pl.BlockSpec((BN, D), lambda r, i: (R * NB + i, 0)),
            pl.BlockSpec((1, D), lambda r, i: (0, 0)),
            pl.BlockSpec((1, D, CP), lambda r, i: (r, 0, 0)),
        ],
        out_specs=pl.BlockSpec((BN, CP), lambda r, i: (r * NB + i, 0)),
        out_shape=jax.ShapeDtypeStruct((nmat * N, CP), _f32),
    )(acc, y0, b, ws)


def _out_body(acc_ref, y1_ref, b_ref, o_ref):
    z = jnp.maximum(acc_ref[0] + acc_ref[1] + y1_ref[...] + b_ref[...], 0.0)
    col = lax.broadcasted_iota(_i32, z.shape, 1)
    msk = col < C
    zm = jnp.where(msk, z, -jnp.inf)
    m = jnp.max(zm, axis=1, keepdims=True)
    s = jnp.sum(jnp.where(msk, jnp.exp(z - m), 0.0), axis=1, keepdims=True)
    o_ref[...] = z - m - jnp.log(s)


def _outk(acc, y1, b):
    return pl.pallas_call(
        _out_body,
        grid=(NB,),
        in_specs=[
            pl.BlockSpec((2, BN, CP), lambda i: (0, i, 0)),
            pl.BlockSpec((BN, CP), lambda i: (R * NB + i, 0)),
            pl.BlockSpec((1, CP), lambda i: (0, 0)),
        ],
        out_specs=pl.BlockSpec((BN, CP), lambda i: (i, 0)),
        out_shape=jax.ShapeDtypeStruct((N, CP), _f32),
    )(acc, y1, b)


def kernel(x, edge_index, edge_type, W_rel0, W_root0, b0, W_rel1, W_root1, b1):
    src = edge_index[0]
    dst = edge_index[1]
    et = edge_type

    zeros_nr = jnp.zeros((ZRT, ROWW), _f32)
    ones_e = jnp.zeros((ECH, ROWW), _f32).at[:, 0].set(1.0)
    part = _hist(dst, et, zeros_nr, ones_e)
    w_e, rid = _prep(src, dst, et, part.reshape(NC, NR))

    W0s = jnp.concatenate([W_rel0, W_root0[None]], axis=0)
    y0 = _mm(x, W0s)
    zeros_d = jnp.zeros((ZCH, D), _f32)
    acc0 = _edge_l1(rid, dst, w_e, y0, zeros_d)

    W1s = jnp.concatenate(
        [jnp.pad(W_rel1, ((0, 0), (0, 0), (0, CP - C))),
         jnp.pad(W_root1, ((0, 0), (0, CP - C)))[None]], axis=0)
    y1 = _l2(acc0, y0, b0.reshape(1, D), W1s)
    zeros_c = jnp.zeros((ZCH, CP), _f32)
    acc1 = _edge_l2(rid, dst, w_e, y1, zeros_c)

    b1p = jnp.pad(b1, (0, CP - C)).reshape(1, CP)
    outp = _outk(acc1, y1, b1p)
    return outp[:, :C]


# speedup vs baseline: 7.8998x; 3.4028x over previous
import functools

import jax
import jax.numpy as jnp
from jax import lax
from jax.experimental import pallas as pl
from jax.experimental.pallas import tpu as pltpu
from jax.experimental.pallas import tpu_sc as plsc

N = 10000
E = 320000
R = 8
D = 128
C = 40
CP = 48
NR = N * R

NC = 2
NS = 16
NW = NC * NS
EPW = E // NW
ECH = 80
NCHUNK = EPW // ECH
RPT = NR // NS
NPT = N // NS
ZCH = 125
ROWW = 16
ZRT = 1000
WIN = 24
PCH = 2000

_mesh = plsc.VectorSubcoreMesh(core_axis_name="c", subcore_axis_name="s",
                               num_cores=NC, num_subcores=NS)
_f32 = jnp.float32
_i32 = jnp.int32
_params = pltpu.CompilerParams(use_tc_tiling_on_sc=False,
                               needs_layout_passes=False)


def _hist_body(dst_h, et_h, zeros_h, ones_h, part_h,
               dstv, etv, segv, onesv, sem, cnt_sh):
    cid = lax.axis_index("c")
    sid = lax.axis_index("s")
    a0 = sid * RPT
    base = (cid * NS + sid) * EPW
    pltpu.sync_copy(dst_h.at[pl.ds(base, EPW)], dstv)
    pltpu.sync_copy(et_h.at[pl.ds(base, EPW)], etv)
    pltpu.sync_copy(ones_h, onesv)
    for k in range(RPT // ZRT):
        pltpu.sync_copy(zeros_h, cnt_sh.at[pl.ds(a0 + k * ZRT, ZRT)])

    def seg_blk(b, carry):
        for j in range(ECH // 16):
            o = b * ECH + j * 16
            segv[b, pl.ds(j * 16, 16)] = (dstv[pl.ds(o, 16)] * R
                                          + etv[pl.ds(o, 16)])
        return carry

    lax.fori_loop(0, NCHUNK, seg_blk, 0)
    plsc.subcore_barrier()

    def blk(b, carry):
        @pl.when(b >= WIN)
        def _():
            pltpu.make_async_copy(onesv, cnt_sh.at[segv.at[0]], sem).wait()
        pltpu.async_copy(onesv, cnt_sh.at[segv.at[b]], sem, add=True)
        return carry

    lax.fori_loop(0, NCHUNK, blk, 0)

    def drain(b, carry):
        pltpu.make_async_copy(onesv, cnt_sh.at[segv.at[0]], sem).wait()
        return carry

    lax.fori_loop(0, WIN, drain, 0)
    plsc.subcore_barrier()
    pltpu.sync_copy(cnt_sh.at[pl.ds(a0, RPT), pl.ds(0, 8)],
                    part_h.at[cid, pl.ds(a0, RPT)])


_hist = pl.kernel(
    _hist_body,
    out_type=jax.ShapeDtypeStruct((NC, NR, 8), _f32),
    mesh=_mesh,
    compiler_params=_params,
    scratch_types=[
        pltpu.VMEM((EPW,), _i32),
        pltpu.VMEM((EPW,), _i32),
        pltpu.VMEM((NCHUNK, ECH), _i32),
        pltpu.VMEM((ECH, ROWW), _f32),
        pltpu.SemaphoreType.DMA,
        pltpu.VMEM_SHARED((NR, ROWW), _f32),
    ],
)


SUB = 1000


def _prep_body(src_h, dst_h, et_h, part_h, w_h, rid_h,
               av, bv, cv, invv, srcv, dstv, etv, wbuf, ridbuf, inv_sh):
    cid = lax.axis_index("c")
    sid = lax.axis_index("s")
    a0 = sid * RPT

    z16 = jnp.zeros((16,), _i32)
    io16 = jnp.arange(16, dtype=_i32)
    for cc in range(RPT // SUB):
        pltpu.sync_copy(part_h.at[0, pl.ds(a0 + cc * SUB, SUB)], av)
        pltpu.sync_copy(part_h.at[1, pl.ds(a0 + cc * SUB, SUB)], bv)

        def inv_blk(k, carry):
            off = jnp.minimum(k * 16, SUB - 16)
            rows16 = off + io16
            cnt16 = (plsc.load_gather(av, [rows16, z16])
                     + plsc.load_gather(bv, [rows16, z16]))
            cv[pl.ds(cc * SUB + off, 16)] = 1.0 / jnp.maximum(cnt16, 1.0)
            return carry

        lax.fori_loop(0, (SUB + 15) // 16, inv_blk, 0)
    pltpu.sync_copy(cv, inv_sh.at[pl.ds(a0, RPT)])
    plsc.subcore_barrier()
    pltpu.sync_copy(inv_sh, invv)

    base = (cid * NS + sid) * EPW

    def stage(c, carry):
        off = base + c * PCH
        pltpu.sync_copy(src_h.at[pl.ds(off, PCH)], srcv)
        pltpu.sync_copy(dst_h.at[pl.ds(off, PCH)], dstv)
        pltpu.sync_copy(et_h.at[pl.ds(off, PCH)], etv)

        def blk(k, c2):
            o = k * 16
            s16 = srcv[pl.ds(o, 16)]
            d16 = dstv[pl.ds(o, 16)]
            t16 = etv[pl.ds(o, 16)]
            seg = d16 * R + t16
            wbuf[pl.ds(o, 16)] = plsc.load_gather(invv, [seg])
            ridbuf[pl.ds(o, 16)] = (t16 * N + s16) + (d16 << 17)
            return c2

        lax.fori_loop(0, PCH // 16, blk, 0, unroll=2)
        pltpu.sync_copy(wbuf, w_h.at[pl.ds(off, PCH)])
        pltpu.sync_copy(ridbuf, rid_h.at[pl.ds(off, PCH)])
        return carry

    lax.fori_loop(0, EPW // PCH, stage, 0)


_prep = pl.kernel(
    _prep_body,
    out_type=(jax.ShapeDtypeStruct((E,), _f32),
              jax.ShapeDtypeStruct((E,), _i32)),
    mesh=_mesh,
    compiler_params=_params,
    scratch_types=[
        pltpu.VMEM((SUB, 8), _f32),
        pltpu.VMEM((SUB, 8), _f32),
        pltpu.VMEM((RPT,), _f32),
        pltpu.VMEM((NR,), _f32),
        pltpu.VMEM((PCH,), _i32),
        pltpu.VMEM((PCH,), _i32),
        pltpu.VMEM((PCH,), _i32),
        pltpu.VMEM((PCH,), _f32),
        pltpu.VMEM((PCH,), _i32),
        pltpu.VMEM_SHARED((NR,), _f32),
    ],
)


NBUF = 3


def _edge_body(F, pk_h, w_h, y_h, zeros_h, acc_h,
               pkv, rid0, rid1, rid2, d0, d1, d2, w0, w1, w2,
               rows0, rows1, rows2, g0, g1, g2, s0, s1, s2, acc_sh):
    cid = lax.axis_index("c")
    sid = lax.axis_index("s")
    wid = cid * NS + sid
    rows = (rows0, rows1, rows2)
    rid_sm = (rid0, rid1, rid2)
    dsm = (d0, d1, d2)
    w_sm = (w0, w1, w2)
    gsem = (g0, g1, g2)
    ssem = (s0, s1, s2)

    pltpu.sync_copy(pk_h.at[wid], pkv)

    def unpack_rid(b, k):
        for j in range(ECH // 16):
            pk16 = pkv[b, pl.ds(j * 16, 16)]
            rid_sm[k][pl.ds(j * 16, 16)] = pk16 & 0x1FFFF

    def unpack_dst(b, k):
        for j in range(ECH // 16):
            pk16 = pkv[b, pl.ds(j * 16, 16)]
            dsm[k][pl.ds(j * 16, 16)] = pk16 >> 17

    def issue_gather(b, k):
        pltpu.async_copy(y_h.at[rid_sm[k]], rows[k], gsem[k])
        pltpu.async_copy(w_h.at[pl.ds(wid * EPW + b * ECH, ECH)],
                         w_sm[k], gsem[k])

    for k in range(2):
        unpack_rid(k, k)
        issue_gather(k, k)
    for k in range(NPT // ZCH):
        pltpu.sync_copy(zeros_h, acc_sh.at[pl.ds(sid * NPT + k * ZCH, ZCH)])
    plsc.subcore_barrier()

    def do_chunk(b, k):
        pltpu.make_async_copy(y_h.at[rid_sm[k]], rows[k], gsem[k]).wait()
        pltpu.make_async_copy(w_h.at[pl.ds(0, ECH)], w_sm[k], gsem[k]).wait()

        def ei(i, c2):
            ws = plsc.load_gather(w_sm[k], [jnp.full((16,), i, _i32)])
            for j in range(F // 16):
                rows[k][i, pl.ds(j * 16, 16)] = (
                    rows[k][i, pl.ds(j * 16, 16)] * ws)
            return c2

        lax.fori_loop(0, ECH, ei, 0, unroll=4)
        unpack_dst(b, k)
        pltpu.async_copy(rows[k], acc_sh.at[dsm[k]], ssem[k], add=True)
        q = b + 2
        kq = (k + 2) % NBUF

        @pl.when(q < NCHUNK)
        def _():
            @pl.when(q >= NBUF)
            def _():
                pltpu.make_async_copy(rows[kq], acc_sh.at[dsm[kq]],
                                      ssem[kq]).wait()
            unpack_rid(q, kq)
            issue_gather(q, kq)

    def g_body(g, carry):
        for k in range(NBUF):
            do_chunk(g * NBUF + k, k)
        return carry

    lax.fori_loop(0, NCHUNK // NBUF, g_body, 0)
    for t in range(NCHUNK % NBUF):
        do_chunk(NCHUNK - (NCHUNK % NBUF) + t, t)
    for k in range(NBUF):
        pltpu.make_async_copy(rows[k], acc_sh.at[dsm[k]], ssem[k]).wait()
    plsc.subcore_barrier()
    pltpu.sync_copy(acc_sh.at[pl.ds(sid * NPT, NPT)],
                    acc_h.at[cid, pl.ds(sid * NPT, NPT)])


def _make_edge(F):
    return pl.kernel(
        functools.partial(_edge_body, F),
        out_type=jax.ShapeDtypeStruct((NC, N, F), _f32),
        mesh=_mesh,
        compiler_params=_params,
        scratch_types=(
            [pltpu.VMEM((NCHUNK, ECH), _i32)]
            + [pltpu.VMEM((ECH,), _i32) for _ in range(3)]
            + [pltpu.VMEM((ECH,), _i32) for _ in range(3)]
            + [pltpu.VMEM((ECH,), _f32) for _ in range(3)]
            + [pltpu.VMEM((ECH, F), _f32) for _ in range(3)]
            + [pltpu.SemaphoreType.DMA for _ in range(6)]
            + [pltpu.VMEM_SHARED((N, F), _f32)]
        ),
    )


_edge_l1 = _make_edge(D)
_edge_l2 = _make_edge(CP)


BN = 2000
NB = N // BN


def _mm_body(x_ref, w_ref, o_ref):
    o_ref[...] = jnp.dot(x_ref[...].astype(jnp.bfloat16),
                         w_ref[0].astype(jnp.bfloat16),
                         preferred_element_type=jnp.float32)


def _mm(x, ws):
    nmat, _, f = ws.shape
    return pl.pallas_call(
        _mm_body,
        grid=(NB, nmat),
        in_specs=[
            pl.BlockSpec((BN, D), lambda i, r: (i, 0)),
            pl.BlockSpec((1, D, f), lambda i, r: (r, 0, 0)),
        ],
        out_specs=pl.BlockSpec((BN, f), lambda i, r: (r * NB + i, 0)),
        out_shape=jax.ShapeDtypeStruct((nmat * N, f), _f32),
    )(x, ws)


def _l2_body(acc_ref, y0_ref, b_ref, w_ref, o_ref):
    h = jnp.maximum(acc_ref[0] + acc_ref[1] + y0_ref[...] + b_ref[...], 0.0)
    o_ref[...] = jnp.dot(h.astype(jnp.bfloat16),
                         w_ref[0].astype(jnp.bfloat16),
                         preferred_element_type=jnp.float32)


def _l2(acc, y0, b, ws):
    nmat = ws.shape[0]
    return pl.pallas_call(
        _l2_body,
        grid=(NB, nmat),
        in_specs=[
            pl.BlockSpec((2, BN, D), lambda i, r: (0, i, 0)),
            pl.BlockSpec((BN, D), lambda i, r: (R * NB + i, 0)),
            pl.BlockSpec((1, D), lambda i, r: (0, 0)),
            pl.BlockSpec((1, D, CP), lambda i, r: (r, 0, 0)),
        ],
        out_specs=pl.BlockSpec((BN, CP), lambda i, r: (r * NB + i, 0)),
        out_shape=jax.ShapeDtypeStruct((nmat * N, CP), _f32),
    )(acc, y0, b, ws)


def _out_body(acc_ref, y1_ref, b_ref, o_ref):
    z = jnp.maximum(acc_ref[0] + acc_ref[1] + y1_ref[...] + b_ref[...], 0.0)
    col = lax.broadcasted_iota(_i32, z.shape, 1)
    msk = col < C
    zm = jnp.where(msk, z, -jnp.inf)
    m = jnp.max(zm, axis=1, keepdims=True)
    s = jnp.sum(jnp.where(msk, jnp.exp(z - m), 0.0), axis=1, keepdims=True)
    o_ref[...] = z - m - jnp.log(s)


def _outk(acc, y1, b):
    return pl.pallas_call(
        _out_body,
        grid=(NB,),
        in_specs=[
            pl.BlockSpec((2, BN, CP), lambda i: (0, i, 0)),
            pl.BlockSpec((BN, CP), lambda i: (R * NB + i, 0)),
            pl.BlockSpec((1, CP), lambda i: (0, 0)),
        ],
        out_specs=pl.BlockSpec((BN, CP), lambda i: (i, 0)),
        out_shape=jax.ShapeDtypeStruct((N, CP), _f32),
    )(acc, y1, b)


def kernel(x, edge_index, edge_type, W_rel0, W_root0, b0, W_rel1, W_root1, b1):
    src = edge_index[0]
    dst = edge_index[1]
    et = edge_type

    zeros_nr = jnp.zeros((ZRT, ROWW), _f32)
    ones_e = jnp.zeros((ECH, ROWW), _f32).at[:, 0].set(1.0)
    part = _hist(dst, et, zeros_nr, ones_e)
    w_e, pk = _prep(src, dst, et, part)

    pk3 = pk.reshape(NW, NCHUNK, ECH)

    W0s = jnp.concatenate([W_rel0, W_root0[None]], axis=0)
    y0 = _mm(x, W0s)
    zeros_d = jnp.zeros((ZCH, D), _f32)
    acc0 = _edge_l1(pk3, w_e, y0, zeros_d)

    W1s = jnp.concatenate(
        [jnp.pad(W_rel1, ((0, 0), (0, 0), (0, CP - C))),
         jnp.pad(W_root1, ((0, 0), (0, CP - C)))[None]], axis=0)
    y1 = _l2(acc0, y0, b0.reshape(1, D), W1s)
    zeros_c = jnp.zeros((ZCH, CP), _f32)
    acc1 = _edge_l2(pk3, w_e, y1, zeros_c)

    b1p = jnp.pad(b1, (0, CP - C)).reshape(1, CP)
    outp = _outk(acc1, y1, b1p)
    return outp[:, :C]
